# baseline (speedup 1.0000x reference)
import jax
import jax.numpy as jnp
from jax.experimental import pallas as pl

_RATIO = 16


def _body(mask_ref, w_ref, emb_ref, gmask_ref):
    B, Sgb, D = emb_ref.shape
    w0 = w_ref[0, :]
    w1 = w_ref[1, :]
    emb_ref[...] = jnp.broadcast_to(w0[None, None, :], (B, Sgb, D))

    @pl.when(pl.program_id(0) == 0)
    def _():
        emb_ref[:, 0, :] = jnp.broadcast_to(w1[None, :], (B, D))

        Bm, Sl = mask_ref.shape
        L = 128
        G = L // _RATIO
        mf = mask_ref[...].astype(jnp.float32).reshape(Bm * Sl // L, L)
        sel = (jax.lax.broadcasted_iota(jnp.int32, (L, G), 0) // _RATIO
               == jax.lax.broadcasted_iota(jnp.int32, (L, G), 1)
               ).astype(jnp.float32)
        s = jax.lax.dot_general(mf, sel, (((1,), (0,)), ((), ())),
                                preferred_element_type=jnp.float32)
        gmask_ref[...] = jnp.where(s == float(_RATIO), 1, 0).astype(jnp.int8)


def kernel(token_ids, padding_mask, embeds_weight):
    B, Sl = padding_mask.shape
    Sg = Sl // _RATIO
    D = embeds_weight.shape[1]
    mask2 = padding_mask.view(jnp.int8)
    sgb = 128
    emb, gmask = pl.pallas_call(
        _body,
        grid=(Sg // sgb,),
        in_specs=[
            pl.BlockSpec((B, Sl), lambda i: (0, 0)),
            pl.BlockSpec((2, D), lambda i: (0, 0)),
        ],
        out_specs=(
            pl.BlockSpec((B, sgb, D), lambda i: (0, i, 0)),
            pl.BlockSpec((B * Sl // 128, 128 // _RATIO), lambda i: (0, 0)),
        ),
        out_shape=(
            jax.ShapeDtypeStruct((B, Sg, D), embeds_weight.dtype),
            jax.ShapeDtypeStruct((B * Sl // 128, 128 // _RATIO), jnp.int8),
        ),
    )(mask2, embeds_weight)
    return (emb, gmask.reshape(B, Sg).view(jnp.bool_))


# speedup vs baseline: 1.0127x; 1.0127x over previous
import jax
import jax.numpy as jnp
from jax.experimental import pallas as pl

_RATIO = 16


def _body(mask_ref, w_ref, emb_ref, gmask_ref):
    B, Sg, D = emb_ref.shape
    w0 = w_ref[0, :]
    w1 = w_ref[1, :]
    emb_ref[...] = jnp.broadcast_to(w0[None, None, :], (B, Sg, D))
    emb_ref[:, 0, :] = jnp.broadcast_to(w1[None, :], (B, D))

    Bm, Sl = mask_ref.shape
    L = 128
    G = L // _RATIO
    mf = mask_ref[...].astype(jnp.float32).reshape(Bm * Sl // L, L)
    sel = (jax.lax.broadcasted_iota(jnp.int32, (L, G), 0) // _RATIO
           == jax.lax.broadcasted_iota(jnp.int32, (L, G), 1)
           ).astype(jnp.float32)
    s = jax.lax.dot_general(mf, sel, (((1,), (0,)), ((), ())),
                            preferred_element_type=jnp.float32)
    gmask_ref[...] = jnp.where(s == float(_RATIO), 1, 0).astype(jnp.int8)


def kernel(token_ids, padding_mask, embeds_weight):
    B, Sl = padding_mask.shape
    Sg = Sl // _RATIO
    D = embeds_weight.shape[1]
    mask2 = padding_mask.view(jnp.int8)
    emb, gmask = pl.pallas_call(
        _body,
        out_shape=(
            jax.ShapeDtypeStruct((B, Sg, D), embeds_weight.dtype),
            jax.ShapeDtypeStruct((B * Sl // 128, 128 // _RATIO), jnp.int8),
        ),
    )(mask2, embeds_weight)
    return (emb, gmask.reshape(B, Sg).view(jnp.bool_))
